# Initial kernel scaffold; baseline (speedup 1.0000x reference)
import jax
import jax.numpy as jnp
from jax.experimental import pallas as pl


def kernel(x, w0, b0, w1, b1, w2, b2, w5, b5, wd0, bd0, wd1, bd1, w710, b710, w912, b912):
    raise NotImplementedError("write your pallas kernel here")


# speedup vs baseline: 8.9988x; 8.9988x over previous
import numpy as np

import jax
import jax.numpy as jnp
from jax.experimental import pallas as pl
from jax.experimental.pallas import tpu as pltpu


def _s2_table(si, so):
    T = np.zeros((9, so * so, si * si), np.float32)
    for kh in range(3):
        for kw in range(3):
            t = kh * 3 + kw
            for r in range(so):
                ir = 2 * r + kh - 1
                if not 0 <= ir < si:
                    continue
                for c in range(so):
                    ic = 2 * c + kw - 1
                    if 0 <= ic < si:
                        T[t, r * so + c, ir * si + ic] = 1.0
    return T


def _s1_table(s):
    T = np.zeros((9, s * s, s * s), np.float32)
    for kh in range(3):
        for kw in range(3):
            t = kh * 3 + kw
            for r in range(s):
                ir = r + kh - 1
                if not 0 <= ir < s:
                    continue
                for c in range(s):
                    ic = c + kw - 1
                    if 0 <= ic < s:
                        T[t, r * s + c, ir * s + ic] = 1.0
    return T


def _s1_up_table():
    T = np.zeros((9, 256, 64), np.float32)
    for kh in range(3):
        for kw in range(3):
            t = kh * 3 + kw
            for r in range(16):
                ir = r + kh - 1
                if not 0 <= ir < 16:
                    continue
                for c in range(16):
                    ic = c + kw - 1
                    if 0 <= ic < 16:
                        T[t, r * 16 + c, (ir // 2) * 8 + (ic // 2)] = 1.0
    return T


def _up4_table():
    U = np.zeros((16, 64), np.float32)
    for r in range(8):
        for c in range(8):
            U[(r // 2) * 4 + (c // 2), r * 8 + c] = 1.0
    return U


_T0 = _s2_table(32, 16)
_T1 = _s2_table(16, 8)
_T2 = _s2_table(8, 4)
_T710 = _s1_table(8)
_T912 = _s1_up_table()
_U4 = _up4_table()
_I64 = np.eye(64, dtype=np.float32)
_I16 = np.eye(16, dtype=np.float32)


def _dense3(w, T):
    t, co, ci = w.shape
    _, hwo, hwi = T.shape
    Tc = jnp.asarray(T)
    m = (w.transpose(0, 2, 1)[:, :, None, :, None]
         * Tc.transpose(0, 2, 1)[:, None, :, None, :]).sum(0)
    return m.reshape(ci * hwi, co * hwo)


def _dense1(w2d, S):
    co, ci = w2d.shape
    p, h = S.shape
    Sc = jnp.asarray(S)
    m = (w2d.T[:, None, :, None] * Sc[None, :, None, :])
    return m.reshape(ci * p, co * h)


def _silu(v):
    return v * pl.reciprocal(1.0 + jnp.exp(-v), approx=True)


def _backbone_kernel(x0_ref, w0_ref, w1_ref, b0_ref, b1_ref, a1_ref):
    f32 = jnp.float32
    a0 = _silu(jnp.dot(x0_ref[...], w0_ref[...], preferred_element_type=f32)
               + b0_ref[...])
    a1_ref[...] = _silu(jnp.dot(a0, w1_ref[...], preferred_element_type=f32)
                        + b1_ref[...])


def _head_kernel(a1_ref, w2_ref, w5a_ref, w5b_ref, wd0_ref, wd1_ref,
                 w710_ref, w912_ref, b2_ref, b5_ref, bd0_ref, bd1_ref,
                 b710_ref, b912_ref, det0_ref, det1_ref, seg_ref):
    f32 = jnp.float32
    a1 = a1_ref[...]
    a2 = _silu(jnp.dot(a1, w2_ref[...], preferred_element_type=f32)
               + b2_ref[...])
    a5 = _silu(jnp.dot(a2, w5a_ref[...], preferred_element_type=f32)
               + jnp.dot(a1, w5b_ref[...], preferred_element_type=f32)
               + b5_ref[...])
    det0_ref[...] = (jnp.dot(a5, wd0_ref[...], preferred_element_type=f32)
                     + bd0_ref[...])
    det1_ref[...] = (jnp.dot(a2, wd1_ref[...], preferred_element_type=f32)
                     + bd1_ref[...])
    a710 = _silu(jnp.dot(a5, w710_ref[...], preferred_element_type=f32)
                 + b710_ref[...])
    seg = (jnp.dot(a710, w912_ref[...], preferred_element_type=f32)
           + b912_ref[...])
    seg_ref[...] = 1.0 / (1.0 + jnp.exp(-seg))


def _const_spec(shape):
    return pl.BlockSpec(shape, lambda b: (0,) * len(shape))


def kernel(x, w0, b0, w1, b1, w2, b2, w5, b5, wd0, bd0, wd1, bd1,
           w710, b710, w912, b912):
    f32 = jnp.float32
    x = x.astype(f32)
    n = x.shape[0]
    bb = 256 if n % 256 == 0 else n
    grid = (n // bb,)

    w0r = w0.reshape(8, 9, 3).transpose(1, 0, 2)
    W0 = _dense3(w0r, _T0)
    W1 = _dense3(w1, _T1)
    W2 = _dense3(w2, _T2)
    W5a = _dense1(w5[:, :32], _U4)
    W5b = _dense1(w5[:, 32:48], _I64)
    Wd0 = _dense1(wd0, _I64)
    Wd1 = _dense1(wd1, _I16)
    W710 = _dense3(w710, _T710)
    W912 = _dense3(w912, _T912)

    def brow(b, rep):
        return jnp.repeat(b.astype(f32), rep)[None, :]

    b0r, b1r, b2r = brow(b0, 256), brow(b1, 64), brow(b2, 16)
    b5r, bd0r, bd1r = brow(b5, 64), brow(bd0, 64), brow(bd1, 16)
    b710r, b912r = brow(b710, 64), brow(b912, 256)

    x0 = x.reshape(n, 3 * 1024)

    a1 = pl.pallas_call(
        _backbone_kernel,
        grid=grid,
        in_specs=[
            pl.BlockSpec((bb, 3072), lambda b: (b, 0)),
            _const_spec((3072, 2048)),
            _const_spec((2048, 1024)),
            _const_spec((1, 2048)),
            _const_spec((1, 1024)),
        ],
        out_specs=pl.BlockSpec((bb, 1024), lambda b: (b, 0)),
        out_shape=jax.ShapeDtypeStruct((n, 1024), f32),
        compiler_params=pltpu.CompilerParams(
            dimension_semantics=("parallel",),
            vmem_limit_bytes=56 * 1024 * 1024),
    )(x0, W0, W1, b0r, b1r)

    det0, det1, seg = pl.pallas_call(
        _head_kernel,
        grid=grid,
        in_specs=[
            pl.BlockSpec((bb, 1024), lambda b: (b, 0)),
            _const_spec((1024, 512)),
            _const_spec((512, 1024)),
            _const_spec((1024, 1024)),
            _const_spec((1024, 2880)),
            _const_spec((512, 720)),
            _const_spec((1024, 1024)),
            _const_spec((1024, 1024)),
            _const_spec((1, 512)),
            _const_spec((1, 1024)),
            _const_spec((1, 2880)),
            _const_spec((1, 720)),
            _const_spec((1, 1024)),
            _const_spec((1, 1024)),
        ],
        out_specs=(
            pl.BlockSpec((bb, 2880), lambda b: (b, 0)),
            pl.BlockSpec((bb, 720), lambda b: (b, 0)),
            pl.BlockSpec((bb, 1024), lambda b: (b, 0)),
        ),
        out_shape=(
            jax.ShapeDtypeStruct((n, 2880), f32),
            jax.ShapeDtypeStruct((n, 720), f32),
            jax.ShapeDtypeStruct((n, 1024), f32),
        ),
        compiler_params=pltpu.CompilerParams(
            dimension_semantics=("parallel",),
            vmem_limit_bytes=56 * 1024 * 1024),
    )(a1, W2, W5a, W5b, Wd0, Wd1, W710, W912,
      b2r, b5r, bd0r, bd1r, b710r, b912r)

    def det_layout(d, ny, nx):
        return jnp.transpose(d.reshape(n, 3, 15, ny, nx), (0, 1, 3, 4, 2))

    det_out = [det_layout(det0, 8, 8), det_layout(det1, 4, 4)]
    seg4 = seg.reshape(n, 4, 16, 16)
    return [det_out, seg4[:, 0:2], seg4[:, 2:4]]
